# Initial kernel scaffold; baseline (speedup 1.0000x reference)
import jax
import jax.numpy as jnp
from jax.experimental import pallas as pl


def kernel(x):
    raise NotImplementedError("write your pallas kernel here")


# speedup vs baseline: 2.0766x; 2.0766x over previous
import functools

import jax
import jax.numpy as jnp
from jax import lax
from jax.experimental import pallas as pl
from jax.experimental.pallas import tpu as pltpu
from jax.experimental.pallas import tpu_sc as plsc

BINS = 256
B, T, C, H, W = 4, 8, 3, 512, 512
SLAB = H * W
NSLAB = B * T * C
NWORKERS = 32
SLABS_PER_WORKER = NSLAB // NWORKERS
CHUNK = 8192
NCHUNK = SLAB // CHUNK
VEC = 16
UNROLL = 8
MAGIC = 2.0 ** 23
MAGIC_INT = 8388608

_N_PER_HIST = float(B * H * W)
_SCALE = 1.0 / ((_N_PER_HIST + 1e-6) * BINS * C * (T - 1))


def _hist_sc_kernel(x_hbm, out_hbm, buf0, buf1, hist, redh, sem0, sem1):
    wid = lax.axis_index("s") * 2 + lax.axis_index("c")

    lane = lax.iota(jnp.int32, VEC)
    lane_off = lane * BINS - MAGIC_INT
    ones = jnp.full((VEC,), 1, jnp.int32)
    zeros = jnp.zeros((VEC,), jnp.int32)

    def do_vec(bufref, off):
        y = bufref[pl.ds(off, VEC)] * 255.0 + MAGIC
        idx = y.astype(jnp.int32) + lane_off
        plsc.addupdate_scatter(hist, [idx], ones)

    def process(bufref):
        def body(j, carry):
            base = j * (VEC * UNROLL)
            for u in range(UNROLL):
                do_vec(bufref, base + u * VEC)
            return carry
        lax.fori_loop(0, CHUNK // (VEC * UNROLL), body, 0)

    def chunk_copy(slab_base, c_idx, bufref, sem):
        src = x_hbm.at[pl.ds(slab_base + c_idx * CHUNK, CHUNK)]
        return pltpu.make_async_copy(src, bufref, sem)

    for i in range(SLABS_PER_WORKER):
        s = wid * SLABS_PER_WORKER + i
        base = s * SLAB

        chunk_copy(base, 0, buf0, sem0).start()
        chunk_copy(base, 1, buf1, sem1).start()

        def zbody(k, carry):
            hist[pl.ds(k * VEC, VEC)] = zeros
            return carry
        lax.fori_loop(0, (VEC * BINS) // VEC, zbody, 0)

        def chunk_body(g, carry, base=base):
            chunk_copy(base, 2 * g, buf0, sem0).wait()
            process(buf0)

            @pl.when(g < (NCHUNK // 2 - 1))
            def _():
                chunk_copy(base, 2 * g + 2, buf0, sem0).start()

            chunk_copy(base, 2 * g + 1, buf1, sem1).wait()
            process(buf1)

            @pl.when(g < (NCHUNK // 2 - 1))
            def _():
                chunk_copy(base, 2 * g + 3, buf1, sem1).start()
            return carry
        lax.fori_loop(0, NCHUNK // 2, chunk_body, 0)

        def red_body(kb, carry):
            o = kb * VEC
            acc = hist[pl.ds(o, VEC)]
            for l in range(1, VEC):
                acc = acc + hist[pl.ds(l * BINS + o, VEC)]
            redh[pl.ds(o, VEC)] = acc
            return carry
        lax.fori_loop(0, BINS // VEC, red_body, 0)

        pltpu.sync_copy(redh, out_hbm.at[pl.ds(s * BINS, BINS)])


_hist_sc = functools.partial(
    pl.kernel,
    mesh=plsc.VectorSubcoreMesh(core_axis_name="c", subcore_axis_name="s"),
    out_type=jax.ShapeDtypeStruct((NSLAB * BINS,), jnp.int32),
    compiler_params=pltpu.CompilerParams(needs_layout_passes=False),
    scratch_types=[
        pltpu.VMEM((CHUNK,), jnp.float32),
        pltpu.VMEM((CHUNK,), jnp.float32),
        pltpu.VMEM((VEC * BINS,), jnp.int32),
        pltpu.VMEM((BINS,), jnp.int32),
        pltpu.SemaphoreType.DMA,
        pltpu.SemaphoreType.DMA,
    ],
)(_hist_sc_kernel)


def _finish_tc_kernel(h_ref, o_ref):
    h = h_ref[...].astype(jnp.float32)
    hs = h[0:24] + h[24:48] + h[48:72] + h[72:96]
    d = jnp.abs(hs[0:21, :] - hs[3:24, :])
    o_ref[0, 0] = jnp.sum(d) * jnp.float32(_SCALE)


def kernel(x):
    flat = x.reshape(-1)
    partials = _hist_sc(flat)
    res = pl.pallas_call(
        _finish_tc_kernel,
        out_shape=jax.ShapeDtypeStruct((1, 1), jnp.float32),
        out_specs=pl.BlockSpec(memory_space=pltpu.SMEM),
    )(partials.reshape(NSLAB, BINS))
    return res[0, 0]


# speedup vs baseline: 2.0774x; 1.0004x over previous
import functools

import jax
import jax.numpy as jnp
from jax import lax
from jax.experimental import pallas as pl
from jax.experimental.pallas import tpu as pltpu
from jax.experimental.pallas import tpu_sc as plsc

BINS = 256
B, T, C, H, W = 4, 8, 3, 512, 512
SLAB = H * W
NSLAB = B * T * C
NWORKERS = 32
SLABS_PER_WORKER = NSLAB // NWORKERS
CHUNK = 8192
NCHUNK = SLAB // CHUNK
VEC = 16
UNROLL = 8
MAGIC = 2.0 ** 23
MAGIC_INT = 8388608
HSTRIDE = BINS + 1
HWORDS = VEC * HSTRIDE

_N_PER_HIST = float(B * H * W)
_SCALE = 1.0 / ((_N_PER_HIST + 1e-6) * BINS * C * (T - 1))


def _hist_sc_kernel(x_hbm, out_hbm, buf0, buf1, hist, redh, sem0, sem1):
    wid = lax.axis_index("s") * 2 + lax.axis_index("c")

    lane = lax.iota(jnp.int32, VEC)
    lane_off = lane * HSTRIDE - MAGIC_INT
    ones = jnp.full((VEC,), 1, jnp.int32)
    zeros = jnp.zeros((VEC,), jnp.int32)

    def do_vec(bufref, off):
        y = bufref[pl.ds(off, VEC)] * 255.0 + MAGIC
        idx = y.astype(jnp.int32) + lane_off
        plsc.addupdate_scatter(hist, [idx], ones)

    def process(bufref):
        def body(j, carry):
            base = j * (VEC * UNROLL)
            for u in range(UNROLL):
                do_vec(bufref, base + u * VEC)
            return carry
        lax.fori_loop(0, CHUNK // (VEC * UNROLL), body, 0)

    def chunk_copy(slab_base, c_idx, bufref, sem):
        src = x_hbm.at[pl.ds(slab_base + c_idx * CHUNK, CHUNK)]
        return pltpu.make_async_copy(src, bufref, sem)

    for i in range(SLABS_PER_WORKER):
        s = wid * SLABS_PER_WORKER + i
        base = s * SLAB

        chunk_copy(base, 0, buf0, sem0).start()
        chunk_copy(base, 1, buf1, sem1).start()

        def zbody(k, carry):
            hist[pl.ds(k * VEC, VEC)] = zeros
            return carry
        lax.fori_loop(0, HWORDS // VEC, zbody, 0)

        def chunk_body(g, carry, base=base):
            chunk_copy(base, 2 * g, buf0, sem0).wait()
            process(buf0)

            @pl.when(g < (NCHUNK // 2 - 1))
            def _():
                chunk_copy(base, 2 * g + 2, buf0, sem0).start()

            chunk_copy(base, 2 * g + 1, buf1, sem1).wait()
            process(buf1)

            @pl.when(g < (NCHUNK // 2 - 1))
            def _():
                chunk_copy(base, 2 * g + 3, buf1, sem1).start()
            return carry
        lax.fori_loop(0, NCHUNK // 2, chunk_body, 0)

        def red_body(kb, carry):
            o = kb * VEC
            acc = hist[pl.ds(o, VEC)]
            for l in range(1, VEC):
                acc = acc + hist[pl.ds(l * HSTRIDE + o, VEC)]
            redh[pl.ds(o, VEC)] = acc
            return carry
        lax.fori_loop(0, BINS // VEC, red_body, 0)

        pltpu.sync_copy(redh, out_hbm.at[pl.ds(s * BINS, BINS)])


_hist_sc = functools.partial(
    pl.kernel,
    mesh=plsc.VectorSubcoreMesh(core_axis_name="c", subcore_axis_name="s"),
    out_type=jax.ShapeDtypeStruct((NSLAB * BINS,), jnp.int32),
    compiler_params=pltpu.CompilerParams(needs_layout_passes=False),
    scratch_types=[
        pltpu.VMEM((CHUNK,), jnp.float32),
        pltpu.VMEM((CHUNK,), jnp.float32),
        pltpu.VMEM((HWORDS,), jnp.int32),
        pltpu.VMEM((BINS,), jnp.int32),
        pltpu.SemaphoreType.DMA,
        pltpu.SemaphoreType.DMA,
    ],
)(_hist_sc_kernel)


def _finish_tc_kernel(h_ref, o_ref):
    h = h_ref[...].astype(jnp.float32)
    hs = h[0:24] + h[24:48] + h[48:72] + h[72:96]
    d = jnp.abs(hs[0:21, :] - hs[3:24, :])
    o_ref[0, 0] = jnp.sum(d) * jnp.float32(_SCALE)


def kernel(x):
    flat = x.reshape(-1)
    partials = _hist_sc(flat)
    res = pl.pallas_call(
        _finish_tc_kernel,
        out_shape=jax.ShapeDtypeStruct((1, 1), jnp.float32),
        out_specs=pl.BlockSpec(memory_space=pltpu.SMEM),
    )(partials.reshape(NSLAB, BINS))
    return res[0, 0]


# speedup vs baseline: 7.7650x; 3.7379x over previous
import functools

import jax
import jax.numpy as jnp
from jax import lax
from jax.experimental import pallas as pl
from jax.experimental.pallas import tpu as pltpu
from jax.experimental.pallas import tpu_sc as plsc

BINS = 256
B, T, C, H, W = 4, 8, 3, 512, 512
SLAB = H * W
NSLAB = B * T * C
NWORKERS = 32
SLABS_PER_WORKER = NSLAB // NWORKERS
CHUNK = 8192
NCHUNK = SLAB // CHUNK
VEC = 16
UNROLL = 8
MAGIC = 2.0 ** 23
MAGIC_INT = 8388608
HSTRIDE = BINS + 1
HWORDS = VEC * HSTRIDE

_N_PER_HIST = float(B * H * W)
_SCALE = 1.0 / ((_N_PER_HIST + 1e-6) * BINS * C * (T - 1))


def _hist_sc_kernel(x_hbm, out_hbm, buf0, buf1, hist, redh, sem0, sem1):
    wid = lax.axis_index("s") * 2 + lax.axis_index("c")

    lane = lax.iota(jnp.int32, VEC)
    lane_off = lane * HSTRIDE - MAGIC_INT
    ones = jnp.full((VEC,), 1, jnp.int32)
    zeros = jnp.zeros((VEC,), jnp.int32)

    def do_vec(bufref, off):
        y = bufref[pl.ds(off, VEC)] * 255.0 + MAGIC
        idx = y.astype(jnp.int32) + lane_off
        plsc.addupdate_scatter(hist, [idx], ones)

    def process(bufref):
        @plsc.parallel_loop(0, CHUNK // VEC, unroll=UNROLL)
        def body(j):
            do_vec(bufref, j * VEC)

    def chunk_copy(slab_base, c_idx, bufref, sem):
        src = x_hbm.at[pl.ds(slab_base + c_idx * CHUNK, CHUNK)]
        return pltpu.make_async_copy(src, bufref, sem)

    for i in range(SLABS_PER_WORKER):
        s = wid * SLABS_PER_WORKER + i
        base = s * SLAB

        chunk_copy(base, 0, buf0, sem0).start()
        chunk_copy(base, 1, buf1, sem1).start()

        def zbody(k, carry):
            hist[pl.ds(k * VEC, VEC)] = zeros
            return carry
        lax.fori_loop(0, HWORDS // VEC, zbody, 0)

        def chunk_body(g, carry, base=base):
            chunk_copy(base, 2 * g, buf0, sem0).wait()
            process(buf0)

            @pl.when(g < (NCHUNK // 2 - 1))
            def _():
                chunk_copy(base, 2 * g + 2, buf0, sem0).start()

            chunk_copy(base, 2 * g + 1, buf1, sem1).wait()
            process(buf1)

            @pl.when(g < (NCHUNK // 2 - 1))
            def _():
                chunk_copy(base, 2 * g + 3, buf1, sem1).start()
            return carry
        lax.fori_loop(0, NCHUNK // 2, chunk_body, 0)

        def red_body(kb, carry):
            o = kb * VEC
            acc = hist[pl.ds(o, VEC)]
            for l in range(1, VEC):
                acc = acc + hist[pl.ds(l * HSTRIDE + o, VEC)]
            redh[pl.ds(o, VEC)] = acc
            return carry
        lax.fori_loop(0, BINS // VEC, red_body, 0)

        pltpu.sync_copy(redh, out_hbm.at[pl.ds(s * BINS, BINS)])


_hist_sc = functools.partial(
    pl.kernel,
    mesh=plsc.VectorSubcoreMesh(core_axis_name="c", subcore_axis_name="s"),
    out_type=jax.ShapeDtypeStruct((NSLAB * BINS,), jnp.int32),
    compiler_params=pltpu.CompilerParams(needs_layout_passes=False),
    scratch_types=[
        pltpu.VMEM((CHUNK,), jnp.float32),
        pltpu.VMEM((CHUNK,), jnp.float32),
        pltpu.VMEM((HWORDS,), jnp.int32),
        pltpu.VMEM((BINS,), jnp.int32),
        pltpu.SemaphoreType.DMA,
        pltpu.SemaphoreType.DMA,
    ],
)(_hist_sc_kernel)


def _finish_tc_kernel(h_ref, o_ref):
    h = h_ref[...].astype(jnp.float32)
    hs = h[0:24] + h[24:48] + h[48:72] + h[72:96]
    d = jnp.abs(hs[0:21, :] - hs[3:24, :])
    o_ref[0, 0] = jnp.sum(d) * jnp.float32(_SCALE)


def kernel(x):
    flat = x.reshape(-1)
    partials = _hist_sc(flat)
    res = pl.pallas_call(
        _finish_tc_kernel,
        out_shape=jax.ShapeDtypeStruct((1, 1), jnp.float32),
        out_specs=pl.BlockSpec(memory_space=pltpu.SMEM),
    )(partials.reshape(NSLAB, BINS))
    return res[0, 0]


# speedup vs baseline: 8.2068x; 1.0569x over previous
import functools

import jax
import jax.numpy as jnp
from jax import lax
from jax.experimental import pallas as pl
from jax.experimental.pallas import tpu as pltpu
from jax.experimental.pallas import tpu_sc as plsc

BINS = 256
B, T, C, H, W = 4, 8, 3, 512, 512
SLAB = H * W
NSLAB = B * T * C
NWORKERS = 32
SLABS_PER_WORKER = NSLAB // NWORKERS
CHUNK = 8192
NCHUNK = SLAB // CHUNK
NBUF = 4
VEC = 16
UNROLL = 8
MAGIC = 2.0 ** 23
MAGIC_INT = 8388608
HSTRIDE = BINS + 1
HWORDS = VEC * HSTRIDE

_N_PER_HIST = float(B * H * W)
_SCALE = 1.0 / ((_N_PER_HIST + 1e-6) * BINS * C * (T - 1))


def _hist_sc_kernel(x_hbm, out_hbm, buf0, buf1, buf2, buf3,
                    hist, redh, sem0, sem1, sem2, sem3):
    bufs = (buf0, buf1, buf2, buf3)
    sems = (sem0, sem1, sem2, sem3)
    wid = lax.axis_index("s") * 2 + lax.axis_index("c")

    lane = lax.iota(jnp.int32, VEC)
    lane_off = lane * HSTRIDE - MAGIC_INT
    ones = jnp.full((VEC,), 1, jnp.int32)
    zeros = jnp.zeros((VEC,), jnp.int32)

    def do_vec(bufref, off):
        y = bufref[pl.ds(off, VEC)] * 255.0 + MAGIC
        idx = y.astype(jnp.int32) + lane_off
        plsc.addupdate_scatter(hist, [idx], ones)

    def process(bufref):
        @plsc.parallel_loop(0, CHUNK // VEC, unroll=UNROLL)
        def body(j):
            do_vec(bufref, j * VEC)

    def chunk_copy(slab_base, c_idx, bufref, sem):
        src = x_hbm.at[pl.ds(slab_base + c_idx * CHUNK, CHUNK)]
        return pltpu.make_async_copy(src, bufref, sem)

    for i in range(SLABS_PER_WORKER):
        s = wid * SLABS_PER_WORKER + i
        base = s * SLAB

        for b in range(NBUF):
            chunk_copy(base, b, bufs[b], sems[b]).start()

        def zbody(k, carry):
            hist[pl.ds(k * VEC, VEC)] = zeros
            return carry
        lax.fori_loop(0, HWORDS // VEC, zbody, 0)

        def chunk_body(g, carry, base=base):
            for b in range(NBUF):
                chunk_copy(base, NBUF * g + b, bufs[b], sems[b]).wait()
                process(bufs[b])

                @pl.when(g < (NCHUNK // NBUF - 1))
                def _(b=b):
                    chunk_copy(base, NBUF * g + b + NBUF, bufs[b],
                               sems[b]).start()
            return carry
        lax.fori_loop(0, NCHUNK // NBUF, chunk_body, 0)

        def red_body(kb, carry):
            o = kb * VEC
            acc = hist[pl.ds(o, VEC)]
            for l in range(1, VEC):
                acc = acc + hist[pl.ds(l * HSTRIDE + o, VEC)]
            redh[pl.ds(o, VEC)] = acc
            return carry
        lax.fori_loop(0, BINS // VEC, red_body, 0)

        pltpu.sync_copy(redh, out_hbm.at[pl.ds(s * BINS, BINS)])


_hist_sc = functools.partial(
    pl.kernel,
    mesh=plsc.VectorSubcoreMesh(core_axis_name="c", subcore_axis_name="s"),
    out_type=jax.ShapeDtypeStruct((NSLAB * BINS,), jnp.int32),
    compiler_params=pltpu.CompilerParams(needs_layout_passes=False),
    scratch_types=(
        [pltpu.VMEM((CHUNK,), jnp.float32) for _ in range(NBUF)]
        + [
            pltpu.VMEM((HWORDS,), jnp.int32),
            pltpu.VMEM((BINS,), jnp.int32),
        ]
        + [pltpu.SemaphoreType.DMA for _ in range(NBUF)]
    ),
)(_hist_sc_kernel)


def _finish_tc_kernel(h_ref, o_ref):
    h = h_ref[...].astype(jnp.float32)
    hs = h[0:24] + h[24:48] + h[48:72] + h[72:96]
    d = jnp.abs(hs[0:21, :] - hs[3:24, :])
    o_ref[0, 0] = jnp.sum(d) * jnp.float32(_SCALE)


def kernel(x):
    flat = x.reshape(-1)
    partials = _hist_sc(flat)
    res = pl.pallas_call(
        _finish_tc_kernel,
        out_shape=jax.ShapeDtypeStruct((1, 1), jnp.float32),
        out_specs=pl.BlockSpec(memory_space=pltpu.SMEM),
    )(partials.reshape(NSLAB, BINS))
    return res[0, 0]


# speedup vs baseline: 14.2048x; 1.7309x over previous
import functools

import jax
import jax.numpy as jnp
from jax import lax
from jax.experimental import pallas as pl
from jax.experimental.pallas import tpu as pltpu
from jax.experimental.pallas import tpu_sc as plsc

BINS = 256
B, T, C, H, W = 4, 8, 3, 512, 512
SLAB = H * W
NSLAB = B * T * C
NWORKERS = 32
SLABS_PER_WORKER = NSLAB // NWORKERS
CROWS = 16
CHUNK = CROWS * W
NCHUNK = SLAB // CHUNK
VPR = W // 16
NBUF = 4
VEC = 16
UNROLL = 8
MAGIC = 2.0 ** 23
MAGIC_INT = 8388608
HSTRIDE = BINS + 1
HWORDS = VEC * HSTRIDE

_N_PER_HIST = float(B * H * W)
_SCALE = 1.0 / ((_N_PER_HIST + 1e-6) * BINS * C * (T - 1))


def _hist_sc_kernel(x_hbm, out_hbm, buf0, buf1, buf2, buf3,
                    hist, redh, sem0, sem1, sem2, sem3):
    bufs = (buf0, buf1, buf2, buf3)
    sems = (sem0, sem1, sem2, sem3)
    wid = lax.axis_index("s") * 2 + lax.axis_index("c")

    lane = lax.iota(jnp.int32, VEC)
    lane_off = lane * HSTRIDE - MAGIC_INT
    ones = jnp.full((VEC,), 1, jnp.int32)
    zeros = jnp.zeros((VEC,), jnp.int32)

    def do_vec(bufref, row, col):
        y = bufref[row, pl.ds(col, VEC)] * 255.0 + MAGIC
        idx = y.astype(jnp.int32) + lane_off
        plsc.addupdate_scatter(hist, [idx], ones)

    def process(bufref):
        @plsc.parallel_loop(0, CHUNK // VEC, unroll=UNROLL)
        def body(j):
            do_vec(bufref, j // VPR, (j % VPR) * VEC)

    def chunk_copy(s, c_idx, bufref, sem):
        src = x_hbm.at[s, pl.ds(c_idx * CROWS, CROWS), :]
        return pltpu.make_async_copy(src, bufref, sem)

    for i in range(SLABS_PER_WORKER):
        s = wid * SLABS_PER_WORKER + i

        for b in range(NBUF):
            chunk_copy(s, b, bufs[b], sems[b]).start()

        def zbody(k, carry):
            hist[pl.ds(k * VEC, VEC)] = zeros
            return carry
        lax.fori_loop(0, HWORDS // VEC, zbody, 0)

        def chunk_body(g, carry, s=s):
            for b in range(NBUF):
                chunk_copy(s, NBUF * g + b, bufs[b], sems[b]).wait()
                process(bufs[b])

                @pl.when(g < (NCHUNK // NBUF - 1))
                def _(b=b):
                    chunk_copy(s, NBUF * g + b + NBUF, bufs[b],
                               sems[b]).start()
            return carry
        lax.fori_loop(0, NCHUNK // NBUF, chunk_body, 0)

        def red_body(kb, carry):
            o = kb * VEC
            acc = hist[pl.ds(o, VEC)]
            for l in range(1, VEC):
                acc = acc + hist[pl.ds(l * HSTRIDE + o, VEC)]
            redh[pl.ds(o, VEC)] = acc
            return carry
        lax.fori_loop(0, BINS // VEC, red_body, 0)

        pltpu.sync_copy(redh, out_hbm.at[pl.ds(s * BINS, BINS)])


_hist_sc = functools.partial(
    pl.kernel,
    mesh=plsc.VectorSubcoreMesh(core_axis_name="c", subcore_axis_name="s"),
    out_type=jax.ShapeDtypeStruct((NSLAB * BINS,), jnp.int32),
    compiler_params=pltpu.CompilerParams(needs_layout_passes=False),
    scratch_types=(
        [pltpu.VMEM((CROWS, W), jnp.float32) for _ in range(NBUF)]
        + [
            pltpu.VMEM((HWORDS,), jnp.int32),
            pltpu.VMEM((BINS,), jnp.int32),
        ]
        + [pltpu.SemaphoreType.DMA for _ in range(NBUF)]
    ),
)(_hist_sc_kernel)


def _finish_tc_kernel(h_ref, o_ref):
    h = h_ref[...].astype(jnp.float32)
    hs = h[0:24] + h[24:48] + h[48:72] + h[72:96]
    d = jnp.abs(hs[0:21, :] - hs[3:24, :])
    o_ref[0, 0] = jnp.sum(d) * jnp.float32(_SCALE)


def kernel(x):
    planes = x.reshape(NSLAB, H, W)
    partials = _hist_sc(planes)
    res = pl.pallas_call(
        _finish_tc_kernel,
        out_shape=jax.ShapeDtypeStruct((1, 1), jnp.float32),
        out_specs=pl.BlockSpec(memory_space=pltpu.SMEM),
    )(partials.reshape(NSLAB, BINS))
    return res[0, 0]


# speedup vs baseline: 14.3988x; 1.0137x over previous
import functools

import jax
import jax.numpy as jnp
from jax import lax
from jax.experimental import pallas as pl
from jax.experimental.pallas import tpu as pltpu
from jax.experimental.pallas import tpu_sc as plsc

BINS = 256
B, T, C, H, W = 4, 8, 3, 512, 512
SLAB = H * W
NSLAB = B * T * C
NWORKERS = 32
SLABS_PER_WORKER = NSLAB // NWORKERS
CROWS = 32
CHUNK = CROWS * W
NCHUNK = SLAB // CHUNK
VPR = W // 16
NBUF = 2
VEC = 16
UNROLL = 8
MAGIC = 2.0 ** 23
MAGIC_INT = 8388608
HSTRIDE = BINS + 1
HWORDS = VEC * HSTRIDE

_N_PER_HIST = float(B * H * W)
_SCALE = 1.0 / ((_N_PER_HIST + 1e-6) * BINS * C * (T - 1))


def _hist_sc_kernel(x_hbm, out_hbm, *scratch):
    bufs = scratch[:NBUF]
    hist, redh = scratch[NBUF:NBUF + 2]
    sems = scratch[NBUF + 2:]
    wid = lax.axis_index("s") * 2 + lax.axis_index("c")

    lane = lax.iota(jnp.int32, VEC)
    lane_off = lane * HSTRIDE - MAGIC_INT
    ones = jnp.full((VEC,), 1, jnp.int32)
    zeros = jnp.zeros((VEC,), jnp.int32)

    def do_vec(bufref, row, col):
        y = bufref[row, pl.ds(col, VEC)] * 255.0 + MAGIC
        idx = y.astype(jnp.int32) + lane_off
        plsc.addupdate_scatter(hist, [idx], ones)

    def process(bufref):
        @plsc.parallel_loop(0, CHUNK // VEC, unroll=UNROLL)
        def body(j):
            do_vec(bufref, j // VPR, (j % VPR) * VEC)

    def chunk_copy(s, c_idx, bufref, sem):
        src = x_hbm.at[s, pl.ds(c_idx * CROWS, CROWS), :]
        return pltpu.make_async_copy(src, bufref, sem)

    for i in range(SLABS_PER_WORKER):
        s = wid * SLABS_PER_WORKER + i

        for b in range(NBUF):
            chunk_copy(s, b, bufs[b], sems[b]).start()

        def zbody(k, carry):
            hist[pl.ds(k * VEC, VEC)] = zeros
            return carry
        lax.fori_loop(0, HWORDS // VEC, zbody, 0)

        def chunk_body(g, carry, s=s):
            for b in range(NBUF):
                chunk_copy(s, NBUF * g + b, bufs[b], sems[b]).wait()
                process(bufs[b])

                @pl.when(g < (NCHUNK // NBUF - 1))
                def _(b=b):
                    chunk_copy(s, NBUF * g + b + NBUF, bufs[b],
                               sems[b]).start()
            return carry
        lax.fori_loop(0, NCHUNK // NBUF, chunk_body, 0)

        def red_body(kb, carry):
            o = kb * VEC
            acc = hist[pl.ds(o, VEC)]
            for l in range(1, VEC):
                acc = acc + hist[pl.ds(l * HSTRIDE + o, VEC)]
            redh[pl.ds(o, VEC)] = acc
            return carry
        lax.fori_loop(0, BINS // VEC, red_body, 0)

        pltpu.sync_copy(redh, out_hbm.at[pl.ds(s * BINS, BINS)])


_hist_sc = functools.partial(
    pl.kernel,
    mesh=plsc.VectorSubcoreMesh(core_axis_name="c", subcore_axis_name="s"),
    out_type=jax.ShapeDtypeStruct((NSLAB * BINS,), jnp.int32),
    compiler_params=pltpu.CompilerParams(needs_layout_passes=False),
    scratch_types=(
        [pltpu.VMEM((CROWS, W), jnp.float32) for _ in range(NBUF)]
        + [
            pltpu.VMEM((HWORDS,), jnp.int32),
            pltpu.VMEM((BINS,), jnp.int32),
        ]
        + [pltpu.SemaphoreType.DMA for _ in range(NBUF)]
    ),
)(_hist_sc_kernel)


def _finish_tc_kernel(h_ref, o_ref):
    h = h_ref[...].astype(jnp.float32)
    hs = h[0:24] + h[24:48] + h[48:72] + h[72:96]
    d = jnp.abs(hs[0:21, :] - hs[3:24, :])
    o_ref[0, 0] = jnp.sum(d) * jnp.float32(_SCALE)


def kernel(x):
    planes = x.reshape(NSLAB, H, W)
    partials = _hist_sc(planes)
    res = pl.pallas_call(
        _finish_tc_kernel,
        out_shape=jax.ShapeDtypeStruct((1, 1), jnp.float32),
        out_specs=pl.BlockSpec(memory_space=pltpu.SMEM),
    )(partials.reshape(NSLAB, BINS))
    return res[0, 0]


# speedup vs baseline: 14.6802x; 1.0195x over previous
import functools

import jax
import jax.numpy as jnp
from jax import lax
from jax.experimental import pallas as pl
from jax.experimental.pallas import tpu as pltpu
from jax.experimental.pallas import tpu_sc as plsc

BINS = 256
B, T, C, H, W = 4, 8, 3, 512, 512
SLAB = H * W
NSLAB = B * T * C
NWORKERS = 32
SLABS_PER_WORKER = NSLAB // NWORKERS
CROWS = 32
CHUNK = CROWS * W
NCHUNK = SLAB // CHUNK
VPR = W // 16
NBUF = 2
VEC = 16
UNROLL = 16
MAGIC = 2.0 ** 23
MAGIC_INT = 8388608
HSTRIDE = BINS + 1
HWORDS = VEC * HSTRIDE

_N_PER_HIST = float(B * H * W)
_SCALE = 1.0 / ((_N_PER_HIST + 1e-6) * BINS * C * (T - 1))


def _hist_sc_kernel(x_hbm, out_hbm, *scratch):
    bufs = scratch[:NBUF]
    hist, redh = scratch[NBUF:NBUF + 2]
    sems = scratch[NBUF + 2:]
    wid = lax.axis_index("s") * 2 + lax.axis_index("c")

    lane = lax.iota(jnp.int32, VEC)
    lane_off = lane * HSTRIDE - MAGIC_INT
    ones = jnp.full((VEC,), 1, jnp.int32)
    zeros = jnp.zeros((VEC,), jnp.int32)

    def do_vec(bufref, row, col):
        y = bufref[row, pl.ds(col, VEC)] * 255.0 + MAGIC
        idx = y.astype(jnp.int32) + lane_off
        plsc.addupdate_scatter(hist, [idx], ones)

    def process(bufref):
        @plsc.parallel_loop(0, CHUNK // VEC, unroll=UNROLL)
        def body(j):
            do_vec(bufref, j // VPR, (j % VPR) * VEC)

    def chunk_copy(s, c_idx, bufref, sem):
        src = x_hbm.at[s, pl.ds(c_idx * CROWS, CROWS), :]
        return pltpu.make_async_copy(src, bufref, sem)

    for i in range(SLABS_PER_WORKER):
        s = wid * SLABS_PER_WORKER + i

        for b in range(NBUF):
            chunk_copy(s, b, bufs[b], sems[b]).start()

        def zbody(k, carry):
            hist[pl.ds(k * VEC, VEC)] = zeros
            return carry
        lax.fori_loop(0, HWORDS // VEC, zbody, 0)

        def chunk_body(g, carry, s=s):
            for b in range(NBUF):
                chunk_copy(s, NBUF * g + b, bufs[b], sems[b]).wait()
                process(bufs[b])

                @pl.when(g < (NCHUNK // NBUF - 1))
                def _(b=b):
                    chunk_copy(s, NBUF * g + b + NBUF, bufs[b],
                               sems[b]).start()
            return carry
        lax.fori_loop(0, NCHUNK // NBUF, chunk_body, 0)

        def red_body(kb, carry):
            o = kb * VEC
            acc = hist[pl.ds(o, VEC)]
            for l in range(1, VEC):
                acc = acc + hist[pl.ds(l * HSTRIDE + o, VEC)]
            redh[pl.ds(o, VEC)] = acc
            return carry
        lax.fori_loop(0, BINS // VEC, red_body, 0)

        pltpu.sync_copy(redh, out_hbm.at[pl.ds(s * BINS, BINS)])


_hist_sc = functools.partial(
    pl.kernel,
    mesh=plsc.VectorSubcoreMesh(core_axis_name="c", subcore_axis_name="s"),
    out_type=jax.ShapeDtypeStruct((NSLAB * BINS,), jnp.int32),
    compiler_params=pltpu.CompilerParams(needs_layout_passes=False),
    scratch_types=(
        [pltpu.VMEM((CROWS, W), jnp.float32) for _ in range(NBUF)]
        + [
            pltpu.VMEM((HWORDS,), jnp.int32),
            pltpu.VMEM((BINS,), jnp.int32),
        ]
        + [pltpu.SemaphoreType.DMA for _ in range(NBUF)]
    ),
)(_hist_sc_kernel)


def _finish_tc_kernel(h_ref, o_ref):
    h = h_ref[...].astype(jnp.float32)
    hs = h[0:24] + h[24:48] + h[48:72] + h[72:96]
    d = jnp.abs(hs[0:21, :] - hs[3:24, :])
    o_ref[0, 0] = jnp.sum(d) * jnp.float32(_SCALE)


def kernel(x):
    planes = x.reshape(NSLAB, H, W)
    partials = _hist_sc(planes)
    res = pl.pallas_call(
        _finish_tc_kernel,
        out_shape=jax.ShapeDtypeStruct((1, 1), jnp.float32),
        out_specs=pl.BlockSpec(memory_space=pltpu.SMEM),
    )(partials.reshape(NSLAB, BINS))
    return res[0, 0]


# speedup vs baseline: 14.8606x; 1.0123x over previous
import functools

import jax
import jax.numpy as jnp
from jax import lax
from jax.experimental import pallas as pl
from jax.experimental.pallas import tpu as pltpu
from jax.experimental.pallas import tpu_sc as plsc

BINS = 256
B, T, C, H, W = 4, 8, 3, 512, 512
SLAB = H * W
NSLAB = B * T * C
NWORKERS = 32
SLABS_PER_WORKER = NSLAB // NWORKERS
CROWS = 32
CHUNK = CROWS * W
NCHUNK = SLAB // CHUNK
VPR = W // 16
NBUF = 2
VEC = 16
UNROLL = 16
MAGIC = 2.0 ** 23
MAGIC_INT = 8388608
HSTRIDE = BINS + 1
HWORDS = VEC * HSTRIDE

_N_PER_HIST = float(B * H * W)
_SCALE = 1.0 / ((_N_PER_HIST + 1e-6) * BINS * C * (T - 1))


def _hist_sc_kernel(x_hbm, out_hbm, *scratch):
    bufs = scratch[:NBUF]
    hist, redh = scratch[NBUF:NBUF + 2]
    sems = scratch[NBUF + 2:]
    wid = lax.axis_index("s") * 2 + lax.axis_index("c")

    lane = lax.iota(jnp.int32, VEC)
    lane_off = lane * HSTRIDE - 0x4B000000
    ones = jnp.full((VEC,), 1, jnp.int32)
    zeros = jnp.zeros((VEC,), jnp.int32)

    def do_vec(bufref, row, col):
        y = bufref[row, pl.ds(col, VEC)] * 255.0 + MAGIC
        idx = plsc.bitcast(y, jnp.int32) + lane_off
        plsc.addupdate_scatter(hist, [idx], ones)

    def process(bufref):
        @plsc.parallel_loop(0, CHUNK // VEC, unroll=UNROLL)
        def body(j):
            do_vec(bufref, j // VPR, (j % VPR) * VEC)

    def chunk_copy(s, c_idx, bufref, sem):
        src = x_hbm.at[s, pl.ds(c_idx * CROWS, CROWS), :]
        return pltpu.make_async_copy(src, bufref, sem)

    for i in range(SLABS_PER_WORKER):
        s = wid * SLABS_PER_WORKER + i

        for b in range(NBUF):
            chunk_copy(s, b, bufs[b], sems[b]).start()

        def zbody(k, carry):
            hist[pl.ds(k * VEC, VEC)] = zeros
            return carry
        lax.fori_loop(0, HWORDS // VEC, zbody, 0)

        def chunk_body(g, carry, s=s):
            for b in range(NBUF):
                chunk_copy(s, NBUF * g + b, bufs[b], sems[b]).wait()
                process(bufs[b])

                @pl.when(g < (NCHUNK // NBUF - 1))
                def _(b=b):
                    chunk_copy(s, NBUF * g + b + NBUF, bufs[b],
                               sems[b]).start()
            return carry
        lax.fori_loop(0, NCHUNK // NBUF, chunk_body, 0)

        def red_body(kb, carry):
            o = kb * VEC
            acc = hist[pl.ds(o, VEC)]
            for l in range(1, VEC):
                acc = acc + hist[pl.ds(l * HSTRIDE + o, VEC)]
            redh[pl.ds(o, VEC)] = acc
            return carry
        lax.fori_loop(0, BINS // VEC, red_body, 0)

        pltpu.sync_copy(redh, out_hbm.at[pl.ds(s * BINS, BINS)])


_hist_sc = functools.partial(
    pl.kernel,
    mesh=plsc.VectorSubcoreMesh(core_axis_name="c", subcore_axis_name="s"),
    out_type=jax.ShapeDtypeStruct((NSLAB * BINS,), jnp.int32),
    compiler_params=pltpu.CompilerParams(needs_layout_passes=False),
    scratch_types=(
        [pltpu.VMEM((CROWS, W), jnp.float32) for _ in range(NBUF)]
        + [
            pltpu.VMEM((HWORDS,), jnp.int32),
            pltpu.VMEM((BINS,), jnp.int32),
        ]
        + [pltpu.SemaphoreType.DMA for _ in range(NBUF)]
    ),
)(_hist_sc_kernel)


def _finish_tc_kernel(h_ref, o_ref):
    h = h_ref[...].astype(jnp.float32)
    hs = h[0:24] + h[24:48] + h[48:72] + h[72:96]
    d = jnp.abs(hs[0:21, :] - hs[3:24, :])
    o_ref[0, 0] = jnp.sum(d) * jnp.float32(_SCALE)


def kernel(x):
    planes = x.reshape(NSLAB, H, W)
    partials = _hist_sc(planes)
    res = pl.pallas_call(
        _finish_tc_kernel,
        out_shape=jax.ShapeDtypeStruct((1, 1), jnp.float32),
        out_specs=pl.BlockSpec(memory_space=pltpu.SMEM),
    )(partials.reshape(NSLAB, BINS))
    return res[0, 0]


# speedup vs baseline: 14.9261x; 1.0044x over previous
import functools

import jax
import jax.numpy as jnp
from jax import lax
from jax.experimental import pallas as pl
from jax.experimental.pallas import tpu as pltpu
from jax.experimental.pallas import tpu_sc as plsc

BINS = 256
B, T, C, H, W = 4, 8, 3, 512, 512
SLAB = H * W
NSLAB = B * T * C
NWORKERS = 32
SLABS_PER_WORKER = NSLAB // NWORKERS
CROWS = 32
CHUNK = CROWS * W
NCHUNK = SLAB // CHUNK
VPR = W // 16
NBUF = 2
VEC = 16
UNROLL = 32
MAGIC = 2.0 ** 23
MAGIC_INT = 8388608
HSTRIDE = BINS + 1
HWORDS = VEC * HSTRIDE

_N_PER_HIST = float(B * H * W)
_SCALE = 1.0 / ((_N_PER_HIST + 1e-6) * BINS * C * (T - 1))


def _hist_sc_kernel(x_hbm, out_hbm, *scratch):
    bufs = scratch[:NBUF]
    hist, redh = scratch[NBUF:NBUF + 2]
    sems = scratch[NBUF + 2:]
    wid = lax.axis_index("s") * 2 + lax.axis_index("c")

    lane = lax.iota(jnp.int32, VEC)
    lane_off = lane * HSTRIDE - 0x4B000000
    ones = jnp.full((VEC,), 1, jnp.int32)
    zeros = jnp.zeros((VEC,), jnp.int32)

    def do_vec(bufref, row, col):
        y = bufref[row, pl.ds(col, VEC)] * 255.0 + MAGIC
        idx = plsc.bitcast(y, jnp.int32) + lane_off
        plsc.addupdate_scatter(hist, [idx], ones)

    def process(bufref):
        @plsc.parallel_loop(0, CHUNK // VEC, unroll=UNROLL)
        def body(j):
            do_vec(bufref, j // VPR, (j % VPR) * VEC)

    def chunk_copy(s, c_idx, bufref, sem):
        src = x_hbm.at[s, pl.ds(c_idx * CROWS, CROWS), :]
        return pltpu.make_async_copy(src, bufref, sem)

    for i in range(SLABS_PER_WORKER):
        s = wid * SLABS_PER_WORKER + i

        for b in range(NBUF):
            chunk_copy(s, b, bufs[b], sems[b]).start()

        def zbody(k, carry):
            hist[pl.ds(k * VEC, VEC)] = zeros
            return carry
        lax.fori_loop(0, HWORDS // VEC, zbody, 0)

        def chunk_body(g, carry, s=s):
            for b in range(NBUF):
                chunk_copy(s, NBUF * g + b, bufs[b], sems[b]).wait()
                process(bufs[b])

                @pl.when(g < (NCHUNK // NBUF - 1))
                def _(b=b):
                    chunk_copy(s, NBUF * g + b + NBUF, bufs[b],
                               sems[b]).start()
            return carry
        lax.fori_loop(0, NCHUNK // NBUF, chunk_body, 0)

        def red_body(kb, carry):
            o = kb * VEC
            acc = hist[pl.ds(o, VEC)]
            for l in range(1, VEC):
                acc = acc + hist[pl.ds(l * HSTRIDE + o, VEC)]
            redh[pl.ds(o, VEC)] = acc
            return carry
        lax.fori_loop(0, BINS // VEC, red_body, 0)

        pltpu.sync_copy(redh, out_hbm.at[pl.ds(s * BINS, BINS)])


_hist_sc = functools.partial(
    pl.kernel,
    mesh=plsc.VectorSubcoreMesh(core_axis_name="c", subcore_axis_name="s"),
    out_type=jax.ShapeDtypeStruct((NSLAB * BINS,), jnp.int32),
    compiler_params=pltpu.CompilerParams(needs_layout_passes=False),
    scratch_types=(
        [pltpu.VMEM((CROWS, W), jnp.float32) for _ in range(NBUF)]
        + [
            pltpu.VMEM((HWORDS,), jnp.int32),
            pltpu.VMEM((BINS,), jnp.int32),
        ]
        + [pltpu.SemaphoreType.DMA for _ in range(NBUF)]
    ),
)(_hist_sc_kernel)


def _finish_tc_kernel(h_ref, o_ref):
    h = h_ref[...].astype(jnp.float32)
    hs = h[0:24] + h[24:48] + h[48:72] + h[72:96]
    d = jnp.abs(hs[0:21, :] - hs[3:24, :])
    o_ref[0, 0] = jnp.sum(d) * jnp.float32(_SCALE)


def kernel(x):
    planes = x.reshape(NSLAB, H, W)
    partials = _hist_sc(planes)
    res = pl.pallas_call(
        _finish_tc_kernel,
        out_shape=jax.ShapeDtypeStruct((1, 1), jnp.float32),
        out_specs=pl.BlockSpec(memory_space=pltpu.SMEM),
    )(partials.reshape(NSLAB, BINS))
    return res[0, 0]
